# baseline (speedup 1.0000x reference)
import jax
import jax.numpy as jnp
from jax import lax
from jax.experimental import pallas as pl
from jax.experimental.pallas import tpu as pltpu
from jax.experimental.pallas import tpu_sc as plsc

_NUM_NODES = 50000
_NUM_EDGES = 800000
_FEAT = 64

_NC = 2
_NS = 16
_NPC = _NUM_NODES // _NC
_RS = 1568
_ACC_ROWS = _NS * _RS
_LAST = _NPC - 15 * _RS
_WB = 196
_ESPAN = _NUM_EDGES // _NS
_CHUNK = 128
_NCH = _ESPAN // _CHUNK
_TAIL = _ESPAN - _NCH * _CHUNK
_NPAIR = _NCH // 2


def _seg_body(dst_hbm, feat_hbm, zeros_hbm, out_hbm,
              acc, f0, f1, d0, d1, i0, i1, it, wbuf, s0, s1, sc0, sc1):
    c = lax.axis_index("c")
    s = lax.axis_index("s")
    lo = c * _NPC
    ebase = s * _ESPAN
    abase = s * _RS

    pltpu.sync_copy(zeros_hbm, wbuf)
    for k in range(_RS // _WB):
        pltpu.sync_copy(wbuf, acc.at[pl.ds(abase + k * _WB, _WB)])
    plsc.subcore_barrier()

    def fire(ch, fbuf, dbuf, sem):
        ch = jnp.minimum(ch, _NCH - 1)
        off = ebase + ch * _CHUNK
        pltpu.async_copy(feat_hbm.at[pl.ds(off, _CHUNK), pl.ds(0, _FEAT)],
                         fbuf, sem)
        pltpu.async_copy(dst_hbm.at[pl.ds(off, _CHUNK)], dbuf, sem)

    def wait(fbuf, dbuf, sem):
        pltpu.make_async_copy(feat_hbm.at[pl.ds(0, _CHUNK), pl.ds(0, _FEAT)],
                              fbuf, sem).wait()
        pltpu.make_async_copy(dst_hbm.at[pl.ds(0, _CHUNK)], dbuf, sem).wait()

    def remap(dref, iref, nvec):
        for j in range(nvec):
            v = dref[pl.ds(j * 16, 16)]
            t = v - lo
            ok = (t >= 0) & (t < _NPC)
            iref[pl.ds(j * 16, 16)] = jnp.where(ok, t, _NPC)

    def scat_fire(fbuf, iref, sem):
        pltpu.async_copy(fbuf, acc.at[iref], sem, add=True)

    def scat_wait(fbuf, iref, sem):
        pltpu.make_async_copy(fbuf, acc.at[iref], sem).wait()

    for j in range(_CHUNK // 16):
        i0[pl.ds(j * 16, 16)] = jnp.full((16,), _NPC, jnp.int32)
        i1[pl.ds(j * 16, 16)] = jnp.full((16,), _NPC, jnp.int32)
    scat_fire(f0, i0, sc0)
    scat_fire(f1, i1, sc1)
    fire(0, f0, d0, s0)

    def pair_body(q, _):
        chb = 2 * q + 1
        wait(f0, d0, s0)
        remap(d0, i0, _CHUNK // 16)
        scat_wait(f1, i1, sc1)
        fire(chb, f1, d1, s1)
        scat_fire(f0, i0, sc0)
        wait(f1, d1, s1)
        remap(d1, i1, _CHUNK // 16)
        scat_wait(f0, i0, sc0)
        fire(chb + 1, f0, d0, s0)
        scat_fire(f1, i1, sc1)
        return ()

    lax.fori_loop(0, _NPAIR, pair_body, ())
    wait(f0, d0, s0)
    scat_wait(f0, i0, sc0)
    scat_wait(f1, i1, sc1)

    toff = ebase + _NCH * _CHUNK
    pltpu.sync_copy(feat_hbm.at[pl.ds(toff, _TAIL), pl.ds(0, _FEAT)],
                    f0.at[pl.ds(0, _TAIL)])
    pltpu.sync_copy(dst_hbm.at[pl.ds(toff, _TAIL)], d0.at[pl.ds(0, _TAIL)])
    for j in range(_TAIL // 16):
        v = d0[pl.ds(j * 16, 16)]
        t = v - lo
        ok = (t >= 0) & (t < _NPC)
        it[pl.ds(j * 16, 16)] = jnp.where(ok, t, _NPC)
    pltpu.sync_copy(f0.at[pl.ds(0, _TAIL)], acc.at[it], add=True)

    plsc.subcore_barrier()

    obase = lo + abase

    def wout(k, rows):
        pltpu.sync_copy(acc.at[pl.ds(abase + k * _WB, rows)],
                        wbuf.at[pl.ds(0, rows)])
        pltpu.sync_copy(wbuf.at[pl.ds(0, rows)],
                        out_hbm.at[pl.ds(obase + k * _WB, rows)])

    for k in range(7):
        wout(k, _WB)

    @pl.when(s < _NS - 1)
    def _():
        wout(7, _WB)

    @pl.when(s == _NS - 1)
    def _():
        wout(7, _LAST - 7 * _WB)


def _tbody(x_ref, o_ref):
    o_ref[:, pl.ds(0, _FEAT)] = x_ref[...].T


@jax.jit
def _run(dst, feat):
    feat_p = pl.pallas_call(
        _tbody,
        grid=(625,),
        in_specs=[pl.BlockSpec((_FEAT, 1280), lambda i: (0, i))],
        out_specs=pl.BlockSpec((1280, 2 * _FEAT), lambda i: (i, 0)),
        out_shape=jax.ShapeDtypeStruct((_NUM_EDGES, 2 * _FEAT), jnp.float32),
    )(feat.T)
    zeros = jnp.zeros((_WB, _FEAT), jnp.float32)
    mesh = plsc.VectorSubcoreMesh(core_axis_name="c", subcore_axis_name="s")
    return pl.kernel(
        _seg_body,
        out_type=jax.ShapeDtypeStruct((_NUM_NODES, _FEAT), jnp.float32),
        mesh=mesh,
        compiler_params=pltpu.CompilerParams(use_tc_tiling_on_sc=False),
        scratch_types=[
            pltpu.VMEM_SHARED((_ACC_ROWS, _FEAT), jnp.float32),
            pltpu.VMEM((_CHUNK, _FEAT), jnp.float32),
            pltpu.VMEM((_CHUNK, _FEAT), jnp.float32),
            pltpu.VMEM((_CHUNK,), jnp.int32),
            pltpu.VMEM((_CHUNK,), jnp.int32),
            pltpu.VMEM((_CHUNK,), jnp.int32),
            pltpu.VMEM((_CHUNK,), jnp.int32),
            pltpu.VMEM((_TAIL,), jnp.int32),
            pltpu.VMEM((_WB, _FEAT), jnp.float32),
            pltpu.SemaphoreType.DMA,
            pltpu.SemaphoreType.DMA,
            pltpu.SemaphoreType.DMA,
            pltpu.SemaphoreType.DMA,
        ],
    )(dst, feat_p, zeros)


def kernel(dst_idx, edge_feat):
    return _run(dst_idx.astype(jnp.int32), edge_feat)


# speedup vs baseline: 1.2149x; 1.2149x over previous
import jax
import jax.numpy as jnp
from jax import lax
from jax.experimental import pallas as pl
from jax.experimental.pallas import tpu as pltpu
from jax.experimental.pallas import tpu_sc as plsc

_NUM_NODES = 50000
_NUM_EDGES = 800000
_FEAT = 64

_NC = 2
_NS = 16
_NPC = _NUM_NODES // _NC
_RS = 1568
_ACC_ROWS = _NS * _RS
_LAST = _NPC - 15 * _RS
_WB = 196
_ESPAN = _NUM_EDGES // _NS
_CHUNK = 128
_NCH = _ESPAN // _CHUNK
_TAIL = _ESPAN - _NCH * _CHUNK
_NPAIR = _NCH // 2


def _seg_body(dst_hbm, feat_hbm, zeros_hbm, out_hbm,
              acc, f0, f1, d0, d1, i0, i1, it, wbuf, s0, s1, sc0, sc1):
    c = lax.axis_index("c")
    s = lax.axis_index("s")
    lo = c * _NPC
    ebase = s * _ESPAN
    abase = s * _RS

    pltpu.sync_copy(zeros_hbm, wbuf)
    for k in range(_RS // _WB):
        pltpu.sync_copy(wbuf, acc.at[pl.ds(abase + k * _WB, _WB)])
    plsc.subcore_barrier()

    def fire(ch, fbuf, dbuf, sem):
        ch = jnp.minimum(ch, _NCH - 1)
        off = ebase + ch * _CHUNK
        pltpu.async_copy(feat_hbm.at[pl.ds(off, _CHUNK), pl.ds(0, _FEAT)],
                         fbuf, sem)
        pltpu.async_copy(dst_hbm.at[pl.ds(off, _CHUNK)], dbuf, sem)

    def wait(fbuf, dbuf, sem):
        pltpu.make_async_copy(feat_hbm.at[pl.ds(0, _CHUNK), pl.ds(0, _FEAT)],
                              fbuf, sem).wait()
        pltpu.make_async_copy(dst_hbm.at[pl.ds(0, _CHUNK)], dbuf, sem).wait()

    def remap(dref, iref, nvec):
        for j in range(nvec):
            v = dref[pl.ds(j * 16, 16)]
            t = v - lo
            ok = (t >= 0) & (t < _NPC)
            iref[pl.ds(j * 16, 16)] = jnp.where(ok, t, _NPC)

    def scat_fire(fbuf, iref, sem):
        pltpu.async_copy(fbuf, acc.at[iref], sem, add=True)

    def scat_wait(fbuf, iref, sem):
        pltpu.make_async_copy(fbuf, acc.at[iref], sem).wait()

    for j in range(_CHUNK // 16):
        i0[pl.ds(j * 16, 16)] = jnp.full((16,), _NPC, jnp.int32)
        i1[pl.ds(j * 16, 16)] = jnp.full((16,), _NPC, jnp.int32)
    scat_fire(f0, i0, sc0)
    scat_fire(f1, i1, sc1)
    fire(0, f0, d0, s0)

    def pair_body(q, _):
        chb = 2 * q + 1
        wait(f0, d0, s0)
        remap(d0, i0, _CHUNK // 16)
        scat_wait(f1, i1, sc1)
        fire(chb, f1, d1, s1)
        scat_fire(f0, i0, sc0)
        wait(f1, d1, s1)
        remap(d1, i1, _CHUNK // 16)
        scat_wait(f0, i0, sc0)
        fire(chb + 1, f0, d0, s0)
        scat_fire(f1, i1, sc1)
        return ()

    lax.fori_loop(0, _NPAIR, pair_body, ())
    wait(f0, d0, s0)
    scat_wait(f0, i0, sc0)
    scat_wait(f1, i1, sc1)

    toff = ebase + _NCH * _CHUNK
    pltpu.sync_copy(feat_hbm.at[pl.ds(toff, _TAIL), pl.ds(0, _FEAT)],
                    f0.at[pl.ds(0, _TAIL)])
    pltpu.sync_copy(dst_hbm.at[pl.ds(toff, _TAIL)], d0.at[pl.ds(0, _TAIL)])
    for j in range(_TAIL // 16):
        v = d0[pl.ds(j * 16, 16)]
        t = v - lo
        ok = (t >= 0) & (t < _NPC)
        it[pl.ds(j * 16, 16)] = jnp.where(ok, t, _NPC)
    pltpu.sync_copy(f0.at[pl.ds(0, _TAIL)], acc.at[it], add=True)

    plsc.subcore_barrier()

    obase = lo + abase

    def wout(k, rows):
        pltpu.sync_copy(acc.at[pl.ds(abase + k * _WB, rows)],
                        wbuf.at[pl.ds(0, rows)])
        pltpu.sync_copy(wbuf.at[pl.ds(0, rows)],
                        out_hbm.at[pl.ds(obase + k * _WB, rows)])

    for k in range(7):
        wout(k, _WB)

    @pl.when(s < _NS - 1)
    def _():
        wout(7, _WB)

    @pl.when(s == _NS - 1)
    def _():
        wout(7, _LAST - 7 * _WB)


def _tbody(x_ref, o_ref):
    o_ref[:, pl.ds(0, _FEAT)] = x_ref[...].T


@jax.jit
def _run(dst, feat):
    feat_p = pl.pallas_call(
        _tbody,
        grid=(250,),
        in_specs=[pl.BlockSpec((_FEAT, 3200), lambda i: (0, i))],
        out_specs=pl.BlockSpec((3200, 2 * _FEAT), lambda i: (i, 0)),
        out_shape=jax.ShapeDtypeStruct((_NUM_EDGES, 2 * _FEAT), jnp.float32),
    )(feat.T)
    zeros = jnp.zeros((_WB, _FEAT), jnp.float32)
    mesh = plsc.VectorSubcoreMesh(core_axis_name="c", subcore_axis_name="s")
    return pl.kernel(
        _seg_body,
        out_type=jax.ShapeDtypeStruct((_NUM_NODES, _FEAT), jnp.float32),
        mesh=mesh,
        compiler_params=pltpu.CompilerParams(use_tc_tiling_on_sc=False),
        scratch_types=[
            pltpu.VMEM_SHARED((_ACC_ROWS, _FEAT), jnp.float32),
            pltpu.VMEM((_CHUNK, _FEAT), jnp.float32),
            pltpu.VMEM((_CHUNK, _FEAT), jnp.float32),
            pltpu.VMEM((_CHUNK,), jnp.int32),
            pltpu.VMEM((_CHUNK,), jnp.int32),
            pltpu.VMEM((_CHUNK,), jnp.int32),
            pltpu.VMEM((_CHUNK,), jnp.int32),
            pltpu.VMEM((_TAIL,), jnp.int32),
            pltpu.VMEM((_WB, _FEAT), jnp.float32),
            pltpu.SemaphoreType.DMA,
            pltpu.SemaphoreType.DMA,
            pltpu.SemaphoreType.DMA,
            pltpu.SemaphoreType.DMA,
        ],
    )(dst, feat_p, zeros)


def kernel(dst_idx, edge_feat):
    return _run(dst_idx.astype(jnp.int32), edge_feat)


# speedup vs baseline: 1.3225x; 1.0886x over previous
import jax
import jax.numpy as jnp
from jax import lax
from jax.experimental import pallas as pl
from jax.experimental.pallas import tpu as pltpu
from jax.experimental.pallas import tpu_sc as plsc

_NUM_NODES = 50000
_NUM_EDGES = 800000
_FEAT = 64

_NC = 2
_NS = 16
_NPC = _NUM_NODES // _NC
_RS = 1568
_ACC_ROWS = _NS * _RS
_LAST = _NPC - 15 * _RS
_WB = 196
_ESPAN = _NUM_EDGES // _NS
_CHUNK = 128
_NCH = _ESPAN // _CHUNK
_TAIL = _ESPAN - _NCH * _CHUNK
_NPAIR = _NCH // 2


def _seg_body(dst_hbm, feat_hbm, zeros_hbm, out_hbm,
              acc, f0, f1, d0, d1, i0, i1, it, wbuf, s0, s1, sc0, sc1):
    c = lax.axis_index("c")
    s = lax.axis_index("s")
    lo = c * _NPC
    ebase = s * _ESPAN
    abase = s * _RS

    pltpu.sync_copy(zeros_hbm, wbuf)
    for k in range(_RS // _WB):
        pltpu.sync_copy(wbuf, acc.at[pl.ds(abase + k * _WB, _WB)])
    plsc.subcore_barrier()

    def fire(ch, fbuf, dbuf, sem):
        ch = jnp.minimum(ch, _NCH - 1)
        off = ebase + ch * _CHUNK
        pltpu.async_copy(feat_hbm.at[pl.ds(off, _CHUNK), pl.ds(0, _FEAT)],
                         fbuf, sem)
        pltpu.async_copy(dst_hbm.at[pl.ds(off, _CHUNK)], dbuf, sem)

    def wait(fbuf, dbuf, sem):
        pltpu.make_async_copy(feat_hbm.at[pl.ds(0, _CHUNK), pl.ds(0, _FEAT)],
                              fbuf, sem).wait()
        pltpu.make_async_copy(dst_hbm.at[pl.ds(0, _CHUNK)], dbuf, sem).wait()

    def remap(dref, iref, nvec):
        for j in range(nvec):
            v = dref[pl.ds(j * 16, 16)]
            t = v - lo
            ok = (t >= 0) & (t < _NPC)
            iref[pl.ds(j * 16, 16)] = jnp.where(ok, t, _NPC)

    def scat_fire(fbuf, iref, sem):
        pltpu.async_copy(fbuf, acc.at[iref], sem, add=True)

    def scat_wait(fbuf, iref, sem):
        pltpu.make_async_copy(fbuf, acc.at[iref], sem).wait()

    for j in range(_CHUNK // 16):
        i0[pl.ds(j * 16, 16)] = jnp.full((16,), _NPC, jnp.int32)
        i1[pl.ds(j * 16, 16)] = jnp.full((16,), _NPC, jnp.int32)
    scat_fire(f0, i0, sc0)
    scat_fire(f1, i1, sc1)
    fire(0, f0, d0, s0)

    def pair_body(q, _):
        chb = 2 * q + 1
        wait(f0, d0, s0)
        remap(d0, i0, _CHUNK // 16)
        scat_wait(f1, i1, sc1)
        fire(chb, f1, d1, s1)
        scat_fire(f0, i0, sc0)
        wait(f1, d1, s1)
        remap(d1, i1, _CHUNK // 16)
        scat_wait(f0, i0, sc0)
        fire(chb + 1, f0, d0, s0)
        scat_fire(f1, i1, sc1)
        return ()

    lax.fori_loop(0, _NPAIR, pair_body, ())
    wait(f0, d0, s0)
    scat_wait(f0, i0, sc0)
    scat_wait(f1, i1, sc1)

    toff = ebase + _NCH * _CHUNK
    pltpu.sync_copy(feat_hbm.at[pl.ds(toff, _TAIL), pl.ds(0, _FEAT)],
                    f0.at[pl.ds(0, _TAIL)])
    pltpu.sync_copy(dst_hbm.at[pl.ds(toff, _TAIL)], d0.at[pl.ds(0, _TAIL)])
    for j in range(_TAIL // 16):
        v = d0[pl.ds(j * 16, 16)]
        t = v - lo
        ok = (t >= 0) & (t < _NPC)
        it[pl.ds(j * 16, 16)] = jnp.where(ok, t, _NPC)
    pltpu.sync_copy(f0.at[pl.ds(0, _TAIL)], acc.at[it], add=True)

    plsc.subcore_barrier()

    obase = lo + abase

    def wout(k, rows):
        pltpu.sync_copy(acc.at[pl.ds(abase + k * _WB, rows)],
                        wbuf.at[pl.ds(0, rows)])
        pltpu.sync_copy(wbuf.at[pl.ds(0, rows)],
                        out_hbm.at[pl.ds(obase + k * _WB, rows)])

    for k in range(7):
        wout(k, _WB)

    @pl.when(s < _NS - 1)
    def _():
        wout(7, _WB)

    @pl.when(s == _NS - 1)
    def _():
        wout(7, _LAST - 7 * _WB)


def _tbody(x_ref, o_ref):
    o_ref[:, pl.ds(0, _FEAT)] = x_ref[...].T


@jax.jit
def _run(dst, feat):
    feat_p = pl.pallas_call(
        _tbody,
        grid=(125,),
        in_specs=[pl.BlockSpec((_FEAT, 6400), lambda i: (0, i))],
        out_specs=pl.BlockSpec((6400, 2 * _FEAT), lambda i: (i, 0)),
        out_shape=jax.ShapeDtypeStruct((_NUM_EDGES, 2 * _FEAT), jnp.float32),
    )(feat.T)
    zeros = jnp.zeros((_WB, _FEAT), jnp.float32)
    mesh = plsc.VectorSubcoreMesh(core_axis_name="c", subcore_axis_name="s")
    return pl.kernel(
        _seg_body,
        out_type=jax.ShapeDtypeStruct((_NUM_NODES, _FEAT), jnp.float32),
        mesh=mesh,
        compiler_params=pltpu.CompilerParams(use_tc_tiling_on_sc=False),
        scratch_types=[
            pltpu.VMEM_SHARED((_ACC_ROWS, _FEAT), jnp.float32),
            pltpu.VMEM((_CHUNK, _FEAT), jnp.float32),
            pltpu.VMEM((_CHUNK, _FEAT), jnp.float32),
            pltpu.VMEM((_CHUNK,), jnp.int32),
            pltpu.VMEM((_CHUNK,), jnp.int32),
            pltpu.VMEM((_CHUNK,), jnp.int32),
            pltpu.VMEM((_CHUNK,), jnp.int32),
            pltpu.VMEM((_TAIL,), jnp.int32),
            pltpu.VMEM((_WB, _FEAT), jnp.float32),
            pltpu.SemaphoreType.DMA,
            pltpu.SemaphoreType.DMA,
            pltpu.SemaphoreType.DMA,
            pltpu.SemaphoreType.DMA,
        ],
    )(dst, feat_p, zeros)


def kernel(dst_idx, edge_feat):
    return _run(dst_idx.astype(jnp.int32), edge_feat)


# speedup vs baseline: 1.3761x; 1.0405x over previous
import jax
import jax.numpy as jnp
from jax import lax
from jax.experimental import pallas as pl
from jax.experimental.pallas import tpu as pltpu
from jax.experimental.pallas import tpu_sc as plsc

_NUM_NODES = 50000
_NUM_EDGES = 800000
_FEAT = 64

_NC = 2
_NS = 16
_NPC = _NUM_NODES // _NC
_RS = 1568
_ACC_ROWS = _NS * _RS
_LAST = _NPC - 15 * _RS
_WB = 196
_ESPAN = _NUM_EDGES // _NS
_CHUNK = 128
_NCH = _ESPAN // _CHUNK
_TAIL = _ESPAN - _NCH * _CHUNK
_NPAIR = _NCH // 2


def _seg_body(dst_hbm, feat_hbm, zeros_hbm, out_hbm,
              acc, f0, f1, d0, d1, i0, i1, it, wbuf, s0, s1, sc0, sc1):
    c = lax.axis_index("c")
    s = lax.axis_index("s")
    lo = c * _NPC
    ebase = s * _ESPAN
    abase = s * _RS

    pltpu.sync_copy(zeros_hbm, wbuf)
    for k in range(_RS // _WB):
        pltpu.sync_copy(wbuf, acc.at[pl.ds(abase + k * _WB, _WB)])
    plsc.subcore_barrier()

    def fire(ch, fbuf, dbuf, sem):
        ch = jnp.minimum(ch, _NCH - 1)
        off = ebase + ch * _CHUNK
        pltpu.async_copy(feat_hbm.at[pl.ds(off, _CHUNK), pl.ds(0, _FEAT)],
                         fbuf, sem)
        pltpu.async_copy(dst_hbm.at[pl.ds(off, _CHUNK)], dbuf, sem)

    def wait(fbuf, dbuf, sem):
        pltpu.make_async_copy(feat_hbm.at[pl.ds(0, _CHUNK), pl.ds(0, _FEAT)],
                              fbuf, sem).wait()
        pltpu.make_async_copy(dst_hbm.at[pl.ds(0, _CHUNK)], dbuf, sem).wait()

    def remap(dref, iref, nvec):
        for j in range(nvec):
            v = dref[pl.ds(j * 16, 16)]
            t = v - lo
            ok = (t >= 0) & (t < _NPC)
            iref[pl.ds(j * 16, 16)] = jnp.where(ok, t, _NPC)

    def scat_fire(fbuf, iref, sem):
        pltpu.async_copy(fbuf, acc.at[iref], sem, add=True)

    def scat_wait(fbuf, iref, sem):
        pltpu.make_async_copy(fbuf, acc.at[iref], sem).wait()

    for j in range(_CHUNK // 16):
        i0[pl.ds(j * 16, 16)] = jnp.full((16,), _NPC, jnp.int32)
        i1[pl.ds(j * 16, 16)] = jnp.full((16,), _NPC, jnp.int32)
    scat_fire(f0, i0, sc0)
    scat_fire(f1, i1, sc1)
    fire(0, f0, d0, s0)

    def pair_body(q, _):
        chb = 2 * q + 1
        wait(f0, d0, s0)
        remap(d0, i0, _CHUNK // 16)
        scat_wait(f1, i1, sc1)
        fire(chb, f1, d1, s1)
        scat_fire(f0, i0, sc0)
        wait(f1, d1, s1)
        remap(d1, i1, _CHUNK // 16)
        scat_wait(f0, i0, sc0)
        fire(chb + 1, f0, d0, s0)
        scat_fire(f1, i1, sc1)
        return ()

    lax.fori_loop(0, _NPAIR, pair_body, ())
    wait(f0, d0, s0)
    scat_wait(f0, i0, sc0)
    scat_wait(f1, i1, sc1)

    toff = ebase + _NCH * _CHUNK
    pltpu.sync_copy(feat_hbm.at[pl.ds(toff, _TAIL), pl.ds(0, _FEAT)],
                    f0.at[pl.ds(0, _TAIL)])
    pltpu.sync_copy(dst_hbm.at[pl.ds(toff, _TAIL)], d0.at[pl.ds(0, _TAIL)])
    for j in range(_TAIL // 16):
        v = d0[pl.ds(j * 16, 16)]
        t = v - lo
        ok = (t >= 0) & (t < _NPC)
        it[pl.ds(j * 16, 16)] = jnp.where(ok, t, _NPC)
    pltpu.sync_copy(f0.at[pl.ds(0, _TAIL)], acc.at[it], add=True)

    plsc.subcore_barrier()

    obase = lo + abase

    def wout(k, rows):
        pltpu.sync_copy(acc.at[pl.ds(abase + k * _WB, rows)],
                        wbuf.at[pl.ds(0, rows)])
        pltpu.sync_copy(wbuf.at[pl.ds(0, rows)],
                        out_hbm.at[pl.ds(obase + k * _WB, rows)])

    for k in range(7):
        wout(k, _WB)

    @pl.when(s < _NS - 1)
    def _():
        wout(7, _WB)

    @pl.when(s == _NS - 1)
    def _():
        wout(7, _LAST - 7 * _WB)


def _tbody(x_ref, o_ref):
    o_ref[:, pl.ds(0, _FEAT)] = x_ref[...].T


@jax.jit
def _run(dst, feat):
    feat_p = pl.pallas_call(
        _tbody,
        grid=(50,),
        in_specs=[pl.BlockSpec((_FEAT, 16000), lambda i: (0, i))],
        out_specs=pl.BlockSpec((16000, 2 * _FEAT), lambda i: (i, 0)),
        out_shape=jax.ShapeDtypeStruct((_NUM_EDGES, 2 * _FEAT), jnp.float32),
    )(feat.T)
    zeros = jnp.zeros((_WB, _FEAT), jnp.float32)
    mesh = plsc.VectorSubcoreMesh(core_axis_name="c", subcore_axis_name="s")
    return pl.kernel(
        _seg_body,
        out_type=jax.ShapeDtypeStruct((_NUM_NODES, _FEAT), jnp.float32),
        mesh=mesh,
        compiler_params=pltpu.CompilerParams(use_tc_tiling_on_sc=False),
        scratch_types=[
            pltpu.VMEM_SHARED((_ACC_ROWS, _FEAT), jnp.float32),
            pltpu.VMEM((_CHUNK, _FEAT), jnp.float32),
            pltpu.VMEM((_CHUNK, _FEAT), jnp.float32),
            pltpu.VMEM((_CHUNK,), jnp.int32),
            pltpu.VMEM((_CHUNK,), jnp.int32),
            pltpu.VMEM((_CHUNK,), jnp.int32),
            pltpu.VMEM((_CHUNK,), jnp.int32),
            pltpu.VMEM((_TAIL,), jnp.int32),
            pltpu.VMEM((_WB, _FEAT), jnp.float32),
            pltpu.SemaphoreType.DMA,
            pltpu.SemaphoreType.DMA,
            pltpu.SemaphoreType.DMA,
            pltpu.SemaphoreType.DMA,
        ],
    )(dst, feat_p, zeros)


def kernel(dst_idx, edge_feat):
    return _run(dst_idx.astype(jnp.int32), edge_feat)


# speedup vs baseline: 1.3852x; 1.0066x over previous
import jax
import jax.numpy as jnp
from jax import lax
from jax.experimental import pallas as pl
from jax.experimental.pallas import tpu as pltpu
from jax.experimental.pallas import tpu_sc as plsc

_NUM_NODES = 50000
_NUM_EDGES = 800000
_FEAT = 64

_NC = 2
_NS = 16
_NPC = _NUM_NODES // _NC
_RS = 1568
_ACC_ROWS = _NS * _RS
_LAST = _NPC - 15 * _RS
_WB = 196
_ESPAN = _NUM_EDGES // _NS
_CHUNK = 128
_NCH = _ESPAN // _CHUNK
_TAIL = _ESPAN - _NCH * _CHUNK
_NPAIR = _NCH // 2


def _seg_body(dst_hbm, feat_hbm, zeros_hbm, out_hbm,
              acc, f0, f1, d0, d1, i0, i1, it, wbuf, s0, s1, sc0, sc1):
    c = lax.axis_index("c")
    s = lax.axis_index("s")
    lo = c * _NPC
    ebase = s * _ESPAN
    abase = s * _RS

    pltpu.sync_copy(zeros_hbm, wbuf)
    for k in range(_RS // _WB):
        pltpu.sync_copy(wbuf, acc.at[pl.ds(abase + k * _WB, _WB)])
    plsc.subcore_barrier()

    def fire(ch, fbuf, dbuf, sem):
        ch = jnp.minimum(ch, _NCH - 1)
        off = ebase + ch * _CHUNK
        pltpu.async_copy(feat_hbm.at[pl.ds(off, _CHUNK), pl.ds(0, _FEAT)],
                         fbuf, sem)
        pltpu.async_copy(dst_hbm.at[pl.ds(off, _CHUNK)], dbuf, sem)

    def wait(fbuf, dbuf, sem):
        pltpu.make_async_copy(feat_hbm.at[pl.ds(0, _CHUNK), pl.ds(0, _FEAT)],
                              fbuf, sem).wait()
        pltpu.make_async_copy(dst_hbm.at[pl.ds(0, _CHUNK)], dbuf, sem).wait()

    def remap(dref, iref, nvec):
        for j in range(nvec):
            v = dref[pl.ds(j * 16, 16)]
            t = v - lo
            ok = (t >= 0) & (t < _NPC)
            iref[pl.ds(j * 16, 16)] = jnp.where(ok, t, _NPC)

    def scat_fire(fbuf, iref, sem):
        pltpu.async_copy(fbuf, acc.at[iref], sem, add=True)

    def scat_wait(fbuf, iref, sem):
        pltpu.make_async_copy(fbuf, acc.at[iref], sem).wait()

    for j in range(_CHUNK // 16):
        i0[pl.ds(j * 16, 16)] = jnp.full((16,), _NPC, jnp.int32)
        i1[pl.ds(j * 16, 16)] = jnp.full((16,), _NPC, jnp.int32)
    scat_fire(f0, i0, sc0)
    scat_fire(f1, i1, sc1)
    fire(0, f0, d0, s0)

    def pair_body(q, _):
        chb = 2 * q + 1
        wait(f0, d0, s0)
        remap(d0, i0, _CHUNK // 16)
        scat_wait(f1, i1, sc1)
        fire(chb, f1, d1, s1)
        scat_fire(f0, i0, sc0)
        wait(f1, d1, s1)
        remap(d1, i1, _CHUNK // 16)
        scat_wait(f0, i0, sc0)
        fire(chb + 1, f0, d0, s0)
        scat_fire(f1, i1, sc1)
        return ()

    lax.fori_loop(0, _NPAIR, pair_body, ())
    wait(f0, d0, s0)
    scat_wait(f0, i0, sc0)
    scat_wait(f1, i1, sc1)

    toff = ebase + _NCH * _CHUNK
    pltpu.sync_copy(feat_hbm.at[pl.ds(toff, _TAIL), pl.ds(0, _FEAT)],
                    f0.at[pl.ds(0, _TAIL)])
    pltpu.sync_copy(dst_hbm.at[pl.ds(toff, _TAIL)], d0.at[pl.ds(0, _TAIL)])
    for j in range(_TAIL // 16):
        v = d0[pl.ds(j * 16, 16)]
        t = v - lo
        ok = (t >= 0) & (t < _NPC)
        it[pl.ds(j * 16, 16)] = jnp.where(ok, t, _NPC)
    pltpu.sync_copy(f0.at[pl.ds(0, _TAIL)], acc.at[it], add=True)

    plsc.subcore_barrier()

    obase = lo + abase

    def wout(k, rows):
        pltpu.sync_copy(acc.at[pl.ds(abase + k * _WB, rows)],
                        wbuf.at[pl.ds(0, rows)])
        pltpu.sync_copy(wbuf.at[pl.ds(0, rows)],
                        out_hbm.at[pl.ds(obase + k * _WB, rows)])

    for k in range(7):
        wout(k, _WB)

    @pl.when(s < _NS - 1)
    def _():
        wout(7, _WB)

    @pl.when(s == _NS - 1)
    def _():
        wout(7, _LAST - 7 * _WB)


def _tbody(x_ref, o_ref):
    o_ref[:, pl.ds(0, _FEAT)] = x_ref[...].T


@jax.jit
def _run(dst, feat):
    feat_p = pl.pallas_call(
        _tbody,
        grid=(25,),
        in_specs=[pl.BlockSpec((_FEAT, 32000), lambda i: (0, i))],
        out_specs=pl.BlockSpec((32000, 2 * _FEAT), lambda i: (i, 0)),
        out_shape=jax.ShapeDtypeStruct((_NUM_EDGES, 2 * _FEAT), jnp.float32),
    )(feat.T)
    zeros = jnp.zeros((_WB, _FEAT), jnp.float32)
    mesh = plsc.VectorSubcoreMesh(core_axis_name="c", subcore_axis_name="s")
    return pl.kernel(
        _seg_body,
        out_type=jax.ShapeDtypeStruct((_NUM_NODES, _FEAT), jnp.float32),
        mesh=mesh,
        compiler_params=pltpu.CompilerParams(use_tc_tiling_on_sc=False),
        scratch_types=[
            pltpu.VMEM_SHARED((_ACC_ROWS, _FEAT), jnp.float32),
            pltpu.VMEM((_CHUNK, _FEAT), jnp.float32),
            pltpu.VMEM((_CHUNK, _FEAT), jnp.float32),
            pltpu.VMEM((_CHUNK,), jnp.int32),
            pltpu.VMEM((_CHUNK,), jnp.int32),
            pltpu.VMEM((_CHUNK,), jnp.int32),
            pltpu.VMEM((_CHUNK,), jnp.int32),
            pltpu.VMEM((_TAIL,), jnp.int32),
            pltpu.VMEM((_WB, _FEAT), jnp.float32),
            pltpu.SemaphoreType.DMA,
            pltpu.SemaphoreType.DMA,
            pltpu.SemaphoreType.DMA,
            pltpu.SemaphoreType.DMA,
        ],
    )(dst, feat_p, zeros)


def kernel(dst_idx, edge_feat):
    return _run(dst_idx.astype(jnp.int32), edge_feat)
